# Initial kernel scaffold; baseline (speedup 1.0000x reference)
import functools

import jax
import jax.numpy as jnp
from jax import lax
from jax.experimental import pallas as pl
from jax.experimental.pallas import tpu as pltpu
from jax.experimental.pallas import tpu_sc as plsc

EMBED_DIM = 16
LANES = 128
NC, NS = 2, 16
NW = NC * NS
CHUNK = 16


def _emb_lookup(table, ids2d):
    nr = ids2d.shape[0]
    rows_per_w = nr // NW
    n_chunks = rows_per_w // CHUNK
    mesh = plsc.VectorSubcoreMesh(core_axis_name="c", subcore_axis_name="s")

    @functools.partial(
        pl.kernel,
        mesh=mesh,
        out_type=jax.ShapeDtypeStruct((nr, LANES, EMBED_DIM), jnp.float32),
        scratch_types=[
            pltpu.VMEM((CHUNK, LANES), jnp.int32),
            pltpu.VMEM((CHUNK, LANES, EMBED_DIM), jnp.float32),
            pltpu.SemaphoreType.DMA,
        ],
    )
    def emb_kernel(table_hbm, idx_hbm, out_hbm, idx_v, rows_v, sem):
        wid = lax.axis_index("s") * NC + lax.axis_index("c")
        base = wid * rows_per_w

        def body(g, carry):
            r0 = base + g * CHUNK
            pltpu.sync_copy(idx_hbm.at[pl.ds(r0, CHUNK)], idx_v)
            handles = [
                pltpu.async_copy(table_hbm.at[idx_v.at[j]], rows_v.at[j], sem)
                for j in range(CHUNK)
            ]
            for h in handles:
                h.wait()
            pltpu.sync_copy(rows_v, out_hbm.at[pl.ds(r0, CHUNK)])
            return carry

        lax.fori_loop(0, n_chunks, body, 0)

    return emb_kernel(table, ids2d)


def kernel(input_ids, table):
    b, s = input_ids.shape
    ids2d = input_ids.reshape(b * s // LANES, LANES).astype(jnp.int32)
    out = _emb_lookup(table, ids2d)
    return out.reshape(b, s, EMBED_DIM)


# speedup vs baseline: 2.4876x; 2.4876x over previous
import functools

import jax
import jax.numpy as jnp
from jax import lax
from jax.experimental import pallas as pl
from jax.experimental.pallas import tpu as pltpu
from jax.experimental.pallas import tpu_sc as plsc

EMBED_DIM = 16
LANES = 128
NC, NS = 2, 16
NW = NC * NS
CHUNK = 16


def _emb_lookup(table, ids2d):
    nr = ids2d.shape[0]
    rows_per_w = nr // NW
    n_chunks = rows_per_w // CHUNK
    mesh = plsc.VectorSubcoreMesh(core_axis_name="c", subcore_axis_name="s")

    @functools.partial(
        pl.kernel,
        mesh=mesh,
        compiler_params=pltpu.CompilerParams(use_tc_tiling_on_sc=False),
        out_type=jax.ShapeDtypeStruct((nr, LANES, EMBED_DIM), jnp.float32),
        scratch_types=[
            pltpu.VMEM((CHUNK, LANES), jnp.int32),
            pltpu.VMEM((CHUNK, LANES, EMBED_DIM), jnp.float32),
            pltpu.SemaphoreType.DMA,
        ],
    )
    def emb_kernel(table_hbm, idx_hbm, out_hbm, idx_v, rows_v, sem):
        wid = lax.axis_index("s") * NC + lax.axis_index("c")
        base = wid * rows_per_w

        def body(g, carry):
            r0 = base + g * CHUNK
            pltpu.sync_copy(idx_hbm.at[pl.ds(r0, CHUNK)], idx_v)
            handles = [
                pltpu.async_copy(table_hbm.at[idx_v.at[j]], rows_v.at[j], sem)
                for j in range(CHUNK)
            ]
            for h in handles:
                h.wait()
            pltpu.sync_copy(rows_v, out_hbm.at[pl.ds(r0, CHUNK)])
            return carry

        lax.fori_loop(0, n_chunks, body, 0)

    return emb_kernel(table, ids2d)


def kernel(input_ids, table):
    b, s = input_ids.shape
    ids2d = input_ids.reshape(b * s // LANES, LANES).astype(jnp.int32)
    out = _emb_lookup(table, ids2d)
    return out.reshape(b, s, EMBED_DIM)


# speedup vs baseline: 2.5307x; 1.0173x over previous
import functools

import jax
import jax.numpy as jnp
from jax import lax
from jax.experimental import pallas as pl
from jax.experimental.pallas import tpu as pltpu
from jax.experimental.pallas import tpu_sc as plsc

EMBED_DIM = 16
LANES = 128
NC, NS = 2, 16
NW = NC * NS
CHUNK = 16


def _emb_lookup(table, ids2d):
    nr = ids2d.shape[0]
    rows_per_w = nr // NW
    n_chunks = rows_per_w // CHUNK
    mesh = plsc.VectorSubcoreMesh(core_axis_name="c", subcore_axis_name="s")

    @functools.partial(
        pl.kernel,
        mesh=mesh,
        compiler_params=pltpu.CompilerParams(use_tc_tiling_on_sc=False),
        out_type=jax.ShapeDtypeStruct((nr, LANES, EMBED_DIM), jnp.float32),
        scratch_types=[
            pltpu.VMEM((2, CHUNK, LANES), jnp.int32),
            pltpu.VMEM((2, CHUNK, LANES, EMBED_DIM), jnp.float32),
            pltpu.SemaphoreType.DMA((2,)),
            pltpu.SemaphoreType.DMA((2,)),
            pltpu.SemaphoreType.DMA((2,)),
        ],
    )
    def emb_kernel(table_hbm, idx_hbm, out_hbm, idx_v, rows_v, isem, gsem, wsem):
        wid = lax.axis_index("s") * NC + lax.axis_index("c")
        base = wid * rows_per_w

        def idx_load(g, slot):
            pltpu.async_copy(
                idx_hbm.at[pl.ds(base + g * CHUNK, CHUNK)],
                idx_v.at[slot],
                isem.at[slot],
            )

        def wait_idx(g, slot):
            pltpu.make_async_copy(
                idx_hbm.at[pl.ds(base + g * CHUNK, CHUNK)],
                idx_v.at[slot],
                isem.at[slot],
            ).wait()

        def fire_gathers(slot):
            for j in range(CHUNK):
                pltpu.async_copy(
                    table_hbm.at[idx_v.at[slot, j]],
                    rows_v.at[slot, j],
                    gsem.at[slot],
                )

        def drain_gathers(g, slot):
            pltpu.make_async_copy(
                out_hbm.at[pl.ds(base + g * CHUNK, CHUNK)],
                rows_v.at[slot],
                gsem.at[slot],
            ).wait()

        def writeback(g, slot):
            pltpu.async_copy(
                rows_v.at[slot],
                out_hbm.at[pl.ds(base + g * CHUNK, CHUNK)],
                wsem.at[slot],
            )

        def wait_writeback(g, slot):
            pltpu.make_async_copy(
                rows_v.at[slot],
                out_hbm.at[pl.ds(base + g * CHUNK, CHUNK)],
                wsem.at[slot],
            ).wait()

        idx_load(0, 0)
        wait_idx(0, 0)
        fire_gathers(0)
        idx_load(1, 1)

        def body(g, carry):
            p = lax.rem(g, 2)
            q = 1 - p
            drain_gathers(g - 1, q)
            writeback(g - 1, q)
            @pl.when(g + 1 < n_chunks)
            def _():
                idx_load(g + 1, q)

            wait_idx(g, p)
            @pl.when(g >= 2)
            def _():
                wait_writeback(g - 2, p)

            fire_gathers(p)
            return carry

        lax.fori_loop(1, n_chunks, body, 0)

        last = n_chunks - 1
        lp = last % 2
        drain_gathers(last, lp)
        writeback(last, lp)
        wait_writeback(last - 1, 1 - lp)
        wait_writeback(last, lp)

    return emb_kernel(table, ids2d)


def kernel(input_ids, table):
    b, s = input_ids.shape
    ids2d = input_ids.reshape(b * s // LANES, LANES).astype(jnp.int32)
    out = _emb_lookup(table, ids2d)
    return out.reshape(b, s, EMBED_DIM)
